# Initial kernel scaffold; baseline (speedup 1.0000x reference)
import jax
import jax.numpy as jnp
from jax.experimental import pallas as pl


def kernel(x, word_table, W_proj, Wg0, bg0, Wt0, bt0, Wg1, bg1, Wt1, bt1):
    raise NotImplementedError("write your pallas kernel here")


# speedup vs baseline: 12.9973x; 12.9973x over previous
import functools

import jax
import jax.numpy as jnp
from jax import lax
from jax.experimental import pallas as pl
from jax.experimental.pallas import tpu as pltpu
from jax.experimental.pallas import tpu_sc as plsc

V, D, H = 100000, 300, 128
B, L = 1024, 200
N = B * L

VBLK = 1000
CH = 128
NW = 32
PER_W = N // NW
NCH = PER_W // CH


def _transform_body(tab_ref, wp_ref, wg0_ref, bg0_ref, wt0_ref, bt0_ref,
                    wg1_ref, bg1_ref, wt1_ref, bt1_ref, out_ref):
    def matT(a, w_ref):
        return lax.dot_general(a, w_ref[...], (((1,), (1,)), ((), ())),
                               preferred_element_type=jnp.float32)

    e = matT(tab_ref[...], wp_ref)
    for wg, bg, wt, bt in ((wg0_ref, bg0_ref, wt0_ref, bt0_ref),
                           (wg1_ref, bg1_ref, wt1_ref, bt1_ref)):
        g = jax.nn.sigmoid(matT(e, wg) + bg[...])
        t = jnp.maximum(matT(e, wt) + bt[...], 0.0)
        e = g * t + (1.0 - g) * e
    out_ref[...] = e


def _transform_table(word_table, W_proj, Wg0, bg0, Wt0, bt0, Wg1, bg1, Wt1, bt1):
    full = lambda shape: pl.BlockSpec(shape, lambda i: (0, 0))
    return pl.pallas_call(
        _transform_body,
        grid=(V // VBLK,),
        in_specs=[
            pl.BlockSpec((VBLK, D), lambda i: (i, 0)),
            full((H, D)),
            full((H, H)), full((1, H)), full((H, H)), full((1, H)),
            full((H, H)), full((1, H)), full((H, H)), full((1, H)),
        ],
        out_specs=pl.BlockSpec((VBLK, H), lambda i: (i, 0)),
        out_shape=jax.ShapeDtypeStruct((V, H), jnp.float32),
    )(word_table, W_proj, Wg0, bg0, Wt0, bt0, Wg1, bg1, Wt1, bt1)


def _gather_body(ft_hbm, idx_hbm, out_hbm, idx_v, r0, r1, s0, s1):
    wid = lax.axis_index("s") * 2 + lax.axis_index("c")
    base = wid * PER_W
    pltpu.sync_copy(idx_hbm.at[pl.ds(base, PER_W)], idx_v)
    rows = (r0, r1)
    sems = (s0, s1)

    def start(c, b):
        pltpu.async_copy(ft_hbm.at[idx_v.at[pl.ds(c * CH, CH)]], rows[b], sems[b])

    def wait(b):
        pltpu.make_async_copy(ft_hbm.at[idx_v.at[pl.ds(0, CH)]],
                              rows[b], sems[b]).wait()

    start(0, 0)
    start(1, 1)

    def body(j, carry):
        for b in range(2):
            c = 2 * j + b
            wait(b)
            pltpu.sync_copy(rows[b], out_hbm.at[pl.ds(base + c * CH, CH)])
            nc = c + 2

            @pl.when(nc < NCH)
            def _():
                start(nc, b)
        return carry

    lax.fori_loop(0, NCH // 2, body, 0)


@functools.partial(
    pl.kernel,
    mesh=plsc.VectorSubcoreMesh(core_axis_name="c", subcore_axis_name="s"),
    out_type=jax.ShapeDtypeStruct((N, H), jnp.float32),
    scratch_types=[
        pltpu.VMEM((PER_W,), jnp.int32),
        pltpu.VMEM((CH, H), jnp.float32),
        pltpu.VMEM((CH, H), jnp.float32),
        pltpu.SemaphoreType.DMA,
        pltpu.SemaphoreType.DMA,
    ],
)
def _gather_rows(ft_hbm, idx_hbm, out_hbm, idx_v, r0, r1, s0, s1):
    _gather_body(ft_hbm, idx_hbm, out_hbm, idx_v, r0, r1, s0, s1)


def kernel(x, word_table, W_proj, Wg0, bg0, Wt0, bt0, Wg1, bg1, Wt1, bt1):
    ftable = _transform_table(word_table, W_proj,
                              Wg0, bg0.reshape(1, H), Wt0, bt0.reshape(1, H),
                              Wg1, bg1.reshape(1, H), Wt1, bt1.reshape(1, H))
    idx = x.reshape(N).astype(jnp.int32)
    out = _gather_rows(ftable, idx)
    return out.reshape(B, L, H)


# speedup vs baseline: 15.2388x; 1.1725x over previous
import functools

import jax
import jax.numpy as jnp
from jax import lax
from jax.experimental import pallas as pl
from jax.experimental.pallas import tpu as pltpu
from jax.experimental.pallas import tpu_sc as plsc

V, D, H = 100000, 300, 128
B, L = 1024, 200
N = B * L

VBLK = 2000
CH = 128
NW = 32
PER_W = N // NW
NCH = PER_W // CH


def _transform_body(tab_ref, wp_ref, wgt0_ref, bgt0_ref, wgt1_ref, bgt1_ref,
                    out_ref):
    def matT(a, w_ref):
        return lax.dot_general(a, w_ref[...], (((1,), (1,)), ((), ())),
                               preferred_element_type=jnp.float32)

    e = matT(tab_ref[...], wp_ref)
    for wgt, bgt in ((wgt0_ref, bgt0_ref), (wgt1_ref, bgt1_ref)):
        z = matT(e, wgt) + bgt[...]
        g = jax.nn.sigmoid(z[:, :H])
        t = jnp.maximum(z[:, H:], 0.0)
        e = g * t + (1.0 - g) * e
    out_ref[...] = e


def _transform_table(word_table, W_proj, Wgt0, bgt0, Wgt1, bgt1):
    full = lambda shape: pl.BlockSpec(shape, lambda i: (0, 0))
    return pl.pallas_call(
        _transform_body,
        grid=(V // VBLK,),
        in_specs=[
            pl.BlockSpec((VBLK, D), lambda i: (i, 0)),
            full((H, D)),
            full((2 * H, H)), full((1, 2 * H)),
            full((2 * H, H)), full((1, 2 * H)),
        ],
        out_specs=pl.BlockSpec((VBLK, H), lambda i: (i, 0)),
        out_shape=jax.ShapeDtypeStruct((V, H), jnp.float32),
    )(word_table, W_proj, Wgt0, bgt0, Wgt1, bgt1)


def _gather_body(ft_hbm, idx_hbm, out_hbm, idx_v, r0, r1, s0, s1):
    wid = lax.axis_index("s") * 2 + lax.axis_index("c")
    base = wid * PER_W
    pltpu.sync_copy(idx_hbm.at[pl.ds(base, PER_W)], idx_v)
    rows = (r0, r1)
    sems = (s0, s1)

    def start(c, b):
        pltpu.async_copy(ft_hbm.at[idx_v.at[pl.ds(c * CH, CH)]], rows[b], sems[b])

    def wait(b):
        pltpu.make_async_copy(ft_hbm.at[idx_v.at[pl.ds(0, CH)]],
                              rows[b], sems[b]).wait()

    start(0, 0)
    start(1, 1)

    def body(j, carry):
        for b in range(2):
            c = 2 * j + b
            wait(b)
            pltpu.sync_copy(rows[b], out_hbm.at[pl.ds(base + c * CH, CH)])
            nc = c + 2

            @pl.when(nc < NCH)
            def _():
                start(nc, b)
        return carry

    lax.fori_loop(0, NCH // 2, body, 0)


@functools.partial(
    pl.kernel,
    mesh=plsc.VectorSubcoreMesh(core_axis_name="c", subcore_axis_name="s"),
    out_type=jax.ShapeDtypeStruct((N, H), jnp.float32),
    scratch_types=[
        pltpu.VMEM((PER_W,), jnp.int32),
        pltpu.VMEM((CH, H), jnp.float32),
        pltpu.VMEM((CH, H), jnp.float32),
        pltpu.SemaphoreType.DMA,
        pltpu.SemaphoreType.DMA,
    ],
)
def _gather_rows(ft_hbm, idx_hbm, out_hbm, idx_v, r0, r1, s0, s1):
    _gather_body(ft_hbm, idx_hbm, out_hbm, idx_v, r0, r1, s0, s1)


def kernel(x, word_table, W_proj, Wg0, bg0, Wt0, bt0, Wg1, bg1, Wt1, bt1):
    ftable = _transform_table(
        word_table, W_proj,
        jnp.concatenate([Wg0, Wt0], axis=0),
        jnp.concatenate([bg0, bt0]).reshape(1, 2 * H),
        jnp.concatenate([Wg1, Wt1], axis=0),
        jnp.concatenate([bg1, bt1]).reshape(1, 2 * H))
    idx = x.reshape(N).astype(jnp.int32)
    out = _gather_rows(ftable, idx)
    return out.reshape(B, L, H)
